# baseline (speedup 1.0000x reference)
import functools

import jax
import jax.numpy as jnp
from jax.experimental import pallas as pl
from jax.experimental.pallas import tpu as pltpu

_TILE = 2048
_GROUP = 16
_NEG_BIG = -1e10
_NEG_PAD = -1e30


def _body(n_g, vocab, tile, group, *refs):
    x_ref = refs[0]
    w_refs = refs[1:1 + group]
    a_refs = refs[1 + group:1 + 2 * group]
    b_refs = refs[1 + 2 * group:1 + 3 * group]
    o_ref, buf_ref, m_ref, s_ref = refs[1 + 3 * group:]
    i = pl.program_id(0)

    @pl.when(i == 0)
    def _init():
        m_ref[...] = jnp.full_like(m_ref, _NEG_PAD)
        s_ref[...] = jnp.zeros_like(s_ref)

    def _group(apply_pad):
        xb = x_ref[...].astype(jnp.bfloat16)
        m_run = m_ref[:, 0:1]
        s_run = s_ref[:, 0:1]
        for q in range(group):
            logits = jax.lax.dot_general(
                xb, w_refs[q][...].astype(jnp.bfloat16),
                (((1,), (0,)), ((), ())),
                preferred_element_type=jnp.float32)
            logits = logits + b_refs[q][...]
            masked = jnp.where(a_refs[q][...] == 0,
                               jnp.float32(_NEG_BIG), logits)
            if apply_pad:
                col = (jax.lax.broadcasted_iota(jnp.int32, masked.shape, 1)
                       + (i * group + q) * tile)
                masked = jnp.where(col < vocab, masked,
                                   jnp.float32(_NEG_PAD))
            buf_ref[i, :, q * tile:(q + 1) * tile] = masked
            m_q = jnp.max(masked, axis=1, keepdims=True)
            m_new = jnp.maximum(m_run, m_q)
            s_run = (s_run * jnp.exp(m_run - m_new)
                     + jnp.sum(jnp.exp(masked - m_new), axis=1,
                               keepdims=True))
            m_run = m_new
        m_ref[:, 0:1] = m_run
        s_ref[:, 0:1] = s_run

    @pl.when(i < n_g - 1)
    def _phase1_clean():
        _group(False)

    @pl.when(i == n_g - 1)
    def _phase1_ragged():
        _group(True)

    @pl.when(i >= n_g)
    def _phase2():
        j = i - n_g
        lse = m_ref[:, 0:1] + jnp.log(s_ref[:, 0:1])
        o_ref[...] = buf_ref[j] - lse


def _build_call(batch, feat, vocab, tile, group):
    n_w = pl.cdiv(vocab, tile)
    n_g = pl.cdiv(vocab, tile * group)
    body = functools.partial(_body, n_g, vocab, tile, group)
    grid = (2 * n_g,)

    def col_idx(q):
        return lambda i: (0, jnp.minimum(i * group + q, n_w - 1))

    in_specs = [pl.BlockSpec((batch, feat), lambda i: (0, 0))]
    in_specs += [pl.BlockSpec((feat, tile), col_idx(q)) for q in range(group)]
    in_specs += [pl.BlockSpec((batch, tile), col_idx(q)) for q in range(group)]
    in_specs += [pl.BlockSpec((1, tile), col_idx(q)) for q in range(group)]
    out_spec = pl.BlockSpec((batch, tile * group),
                            lambda i: (0, jnp.maximum(i - n_g, 0)))
    scratch = [
        pltpu.VMEM((n_g, batch, tile * group), jnp.float32),
        pltpu.VMEM((batch, 128), jnp.float32),
        pltpu.VMEM((batch, 128), jnp.float32),
    ]
    return pl.pallas_call(
        body,
        grid=grid,
        in_specs=in_specs,
        out_specs=out_spec,
        out_shape=jax.ShapeDtypeStruct((batch, vocab), jnp.float32),
        scratch_shapes=scratch,
        compiler_params=pltpu.CompilerParams(
            vmem_limit_bytes=63 * 1024 * 1024),
    )


def kernel(x, available_actions, W, b):
    batch, feat = x.shape
    vocab = W.shape[0]
    if available_actions.ndim == 1:
        available_actions = available_actions[None, :]
    available_actions = jnp.broadcast_to(available_actions, (batch, vocab))
    b2 = b.reshape(1, vocab)
    Wt = W.T
    call = _build_call(batch, feat, vocab, _TILE, _GROUP)
    args = ([x] + [Wt] * _GROUP + [available_actions] * _GROUP
            + [b2] * _GROUP)
    return call(*args)


# speedup vs baseline: 1.0923x; 1.0923x over previous
import functools

import jax
import jax.numpy as jnp
from jax.experimental import pallas as pl
from jax.experimental.pallas import tpu as pltpu

_TILE = 8192
_GROUP = 4
_NEG_BIG = -1e10
_NEG_PAD = -1e30


def _body(n_g, vocab, tile, group, *refs):
    x_ref = refs[0]
    w_refs = refs[1:1 + group]
    a_refs = refs[1 + group:1 + 2 * group]
    b_refs = refs[1 + 2 * group:1 + 3 * group]
    o_ref, buf_ref, m_ref, s_ref = refs[1 + 3 * group:]
    i = pl.program_id(0)

    @pl.when(i == 0)
    def _init():
        m_ref[...] = jnp.full_like(m_ref, _NEG_PAD)
        s_ref[...] = jnp.zeros_like(s_ref)

    def _group(apply_pad):
        xb = x_ref[...].astype(jnp.bfloat16)
        m_run = m_ref[:, 0:1]
        s_run = s_ref[:, 0:1]
        for q in range(group):
            logits = jax.lax.dot_general(
                xb, w_refs[q][...].astype(jnp.bfloat16),
                (((1,), (0,)), ((), ())),
                preferred_element_type=jnp.float32)
            logits = logits + b_refs[q][...]
            masked = jnp.where(a_refs[q][...] == 0,
                               jnp.float32(_NEG_BIG), logits)
            if apply_pad:
                col = (jax.lax.broadcasted_iota(jnp.int32, masked.shape, 1)
                       + (i * group + q) * tile)
                masked = jnp.where(col < vocab, masked,
                                   jnp.float32(_NEG_PAD))
            buf_ref[i, :, q * tile:(q + 1) * tile] = masked
            m_q = jnp.max(masked, axis=1, keepdims=True)
            m_new = jnp.maximum(m_run, m_q)
            s_run = (s_run * jnp.exp(m_run - m_new)
                     + jnp.sum(jnp.exp(masked - m_new), axis=1,
                               keepdims=True))
            m_run = m_new
        m_ref[:, 0:1] = m_run
        s_ref[:, 0:1] = s_run

    @pl.when(i < n_g - 1)
    def _phase1_clean():
        _group(False)

    @pl.when(i == n_g - 1)
    def _phase1_ragged():
        _group(True)

    @pl.when(i >= n_g)
    def _phase2():
        j = i - n_g
        lse = m_ref[:, 0:1] + jnp.log(s_ref[:, 0:1])
        o_ref[...] = buf_ref[j] - lse


def _build_call(batch, feat, vocab, tile, group):
    n_w = pl.cdiv(vocab, tile)
    n_g = pl.cdiv(vocab, tile * group)
    body = functools.partial(_body, n_g, vocab, tile, group)
    grid = (2 * n_g,)

    def col_idx(q):
        return lambda i: (0, jnp.minimum(i * group + q, n_w - 1))

    in_specs = [pl.BlockSpec((batch, feat), lambda i: (0, 0))]
    in_specs += [pl.BlockSpec((feat, tile), col_idx(q)) for q in range(group)]
    in_specs += [pl.BlockSpec((batch, tile), col_idx(q)) for q in range(group)]
    in_specs += [pl.BlockSpec((1, tile), col_idx(q)) for q in range(group)]
    out_spec = pl.BlockSpec((batch, tile * group),
                            lambda i: (0, jnp.maximum(i - n_g, 0)))
    scratch = [
        pltpu.VMEM((n_g, batch, tile * group), jnp.float32),
        pltpu.VMEM((batch, 128), jnp.float32),
        pltpu.VMEM((batch, 128), jnp.float32),
    ]
    return pl.pallas_call(
        body,
        grid=grid,
        in_specs=in_specs,
        out_specs=out_spec,
        out_shape=jax.ShapeDtypeStruct((batch, vocab), jnp.float32),
        scratch_shapes=scratch,
        compiler_params=pltpu.CompilerParams(
            vmem_limit_bytes=63 * 1024 * 1024),
    )


def kernel(x, available_actions, W, b):
    batch, feat = x.shape
    vocab = W.shape[0]
    if available_actions.ndim == 1:
        available_actions = available_actions[None, :]
    available_actions = jnp.broadcast_to(available_actions, (batch, vocab))
    b2 = b.reshape(1, vocab)
    Wt = W.T
    call = _build_call(batch, feat, vocab, _TILE, _GROUP)
    args = ([x] + [Wt] * _GROUP + [available_actions] * _GROUP
            + [b2] * _GROUP)
    return call(*args)
